# baseline (speedup 1.0000x reference)
import functools

import jax
import jax.numpy as jnp
from jax import lax
from jax.experimental import pallas as pl
from jax.experimental.pallas import tpu as pltpu
from jax.experimental.pallas import tpu_sc as plsc

DIM = 128
G = 128
NBUF = 5
WAVES = 4


@functools.cache
def _build(total, nc, ns):
    nw = nc * ns
    rows_total = total // G
    rows_per_w = rows_total // nw
    spi = NBUF * WAVES

    mesh = plsc.VectorSubcoreMesh(core_axis_name="c", subcore_axis_name="s")

    @functools.partial(
        pl.kernel,
        mesh=mesh,
        out_type=jax.ShapeDtypeStruct((total, DIM), jnp.float32),
        scratch_types=[
            pltpu.VMEM((rows_per_w, G), jnp.int32),
            pltpu.VMEM((NBUF, G, DIM), jnp.float32),
            pltpu.VMEM_SHARED((ns, G, DIM), jnp.float32),
            pltpu.SemaphoreType.DMA,
        ] + [pltpu.SemaphoreType.DMA] * NBUF,
    )
    def body(tbl_hbm, idx_hbm, out_hbm, idx_v, rows_v, sp_v, dsem, *gsems):
        sid = lax.axis_index("s")
        wid = sid * nc + lax.axis_index("c")
        row0 = wid * rows_per_w

        pltpu.sync_copy(idx_hbm.at[pl.ds(row0, rows_per_w)], idx_v)

        def fire(step, b):
            return pltpu.async_copy(
                tbl_hbm.at[idx_v.at[step]], rows_v.at[b], gsems[b]
            )

        def outer(gg, _):
            s0 = gg * spi
            gh = [fire(s0 + b, b) for b in range(NBUF)]
            dh = None
            for r in range(spi):
                b = r % NBUF
                gh[b].wait()
                if dh is not None:
                    dh.wait()
                pltpu.sync_copy(rows_v.at[b], sp_v.at[sid])
                dh = pltpu.async_copy(
                    sp_v.at[sid], out_hbm.at[pl.ds((row0 + s0 + r) * G, G)], dsem
                )
                if r + NBUF < spi:
                    gh[b] = fire(s0 + r + NBUF, b)
            dh.wait()
            return _

        lax.fori_loop(0, rows_per_w // spi, outer, 0)

    return body


def kernel(X, table):
    batch, hist = X.shape
    total = batch * hist
    info = plsc.get_sparse_core_info()
    idx = X.reshape(total // G, G).astype(jnp.int32)
    body = _build(total, info.num_cores, info.num_subcores)
    out = body(table, idx)
    return out.reshape(batch, hist, DIM)


# speedup vs baseline: 1.0743x; 1.0743x over previous
import functools

import jax
import jax.numpy as jnp
from jax import lax
from jax.experimental import pallas as pl
from jax.experimental.pallas import tpu as pltpu
from jax.experimental.pallas import tpu_sc as plsc

DIM = 128
G = 128
NBUF = 5
WAVES = 4


@functools.cache
def _build(total, nc, ns):
    nw = nc * ns
    rows_total = total // G
    rows_per_w = rows_total // nw
    spi = NBUF * WAVES

    mesh = plsc.VectorSubcoreMesh(core_axis_name="c", subcore_axis_name="s")

    @functools.partial(
        pl.kernel,
        mesh=mesh,
        out_type=jax.ShapeDtypeStruct((total, DIM), jnp.float32),
        scratch_types=[
            pltpu.VMEM((rows_per_w, G), jnp.int32),
            pltpu.VMEM((NBUF, G, DIM), jnp.float32),
            pltpu.VMEM_SHARED((ns, 2, G // 2, DIM), jnp.float32),
            pltpu.SemaphoreType.DMA,
            pltpu.SemaphoreType.DMA,
        ] + [pltpu.SemaphoreType.DMA] * NBUF,
    )
    def body(tbl_hbm, idx_hbm, out_hbm, idx_v, rows_v, sp_v, dsem0, dsem1, *gsems):
        dsems = (dsem0, dsem1)
        sid = lax.axis_index("s")
        wid = sid * nc + lax.axis_index("c")
        row0 = wid * rows_per_w

        pltpu.sync_copy(idx_hbm.at[pl.ds(row0, rows_per_w)], idx_v)

        def fire(step, b):
            return pltpu.async_copy(
                tbl_hbm.at[idx_v.at[step]], rows_v.at[b], gsems[b]
            )

        def outer(gg, _):
            s0 = gg * spi
            gh = [fire(s0 + b, b) for b in range(NBUF)]
            dh = [None, None]
            for r in range(spi):
                b = r % NBUF
                gh[b].wait()
                for sl in range(2):
                    if dh[sl] is not None:
                        dh[sl].wait()
                    pltpu.sync_copy(
                        rows_v.at[b, pl.ds(sl * (G // 2), G // 2)], sp_v.at[sid, sl]
                    )
                    dh[sl] = pltpu.async_copy(
                        sp_v.at[sid, sl],
                        out_hbm.at[pl.ds((row0 + s0 + r) * G + sl * (G // 2), G // 2)],
                        dsems[sl],
                    )
                if r + NBUF < spi:
                    gh[b] = fire(s0 + r + NBUF, b)
            for sl in range(2):
                dh[sl].wait()
            return _

        lax.fori_loop(0, rows_per_w // spi, outer, 0)

    return body


def kernel(X, table):
    batch, hist = X.shape
    total = batch * hist
    info = plsc.get_sparse_core_info()
    idx = X.reshape(total // G, G).astype(jnp.int32)
    body = _build(total, info.num_cores, info.num_subcores)
    out = body(table, idx)
    return out.reshape(batch, hist, DIM)
